# baseline (speedup 1.0000x reference)
import functools

import jax
import jax.numpy as jnp
from jax import lax
from jax.experimental import pallas as pl
from jax.experimental.pallas import tpu as pltpu
from jax.experimental.pallas import tpu_sc as plsc

NC = 2
NS = 16
LANES = 16
ZR = 80


def _sc_aggregate(x2, src5, dst5, n_nodes, n_edges, wc, edge_split):
    K = src5.shape[4]
    NBUF = src5.shape[3]
    n_rounds = src5.shape[2]
    rpt = n_nodes // NS
    assert n_nodes % NS == 0 and rpt % K == 0

    mesh = plsc.VectorSubcoreMesh(core_axis_name="c", subcore_axis_name="s")

    scratch = (
        pltpu.VMEM_SHARED((n_nodes, wc), jnp.float32),
        pltpu.VMEM((2, NBUF, K), jnp.int32),
        pltpu.VMEM((2, NBUF, K), jnp.int32),
        [pltpu.VMEM((K, wc), jnp.float32) for _ in range(NBUF)],
        [pltpu.SemaphoreType.DMA for _ in range(NBUF)],
        [pltpu.SemaphoreType.DMA for _ in range(NBUF)],
        pltpu.SemaphoreType.DMA,
        pltpu.SemaphoreType.DMA,
    )

    @functools.partial(
        pl.kernel,
        out_type=jax.ShapeDtypeStruct((NC * n_nodes, wc), jnp.float32),
        mesh=mesh, scratch_types=scratch,
        compiler_params=pltpu.CompilerParams(use_tc_tiling_on_sc=False))
    def agg_kernel(src_h, dst_h, x_h, out_h,
                   acc, sidx_v, didx_v, rows, gsems, ssems, sisem, disem):
        c = lax.axis_index("c")
        s = lax.axis_index("s")

        zvec = jnp.zeros((LANES,), jnp.float32)

        def fill_zero(r, _):
            for k in range(wc // LANES):
                rows[0][r, pl.ds(k * LANES, LANES)] = zvec
            return 0
        lax.fori_loop(0, K, fill_zero, 0)

        def zero_acc(j, _):
            pltpu.sync_copy(rows[0], acc.at[pl.ds(s * rpt + j * K, K)])
            return 0
        lax.fori_loop(0, rpt // K, zero_acc, 0)

        plsc.subcore_barrier()

        def idx_start(r, par):
            pltpu.async_copy(src_h.at[c, s, r], sidx_v.at[par], sisem)
            pltpu.async_copy(dst_h.at[c, s, r], didx_v.at[par], disem)

        def idx_wait():
            pltpu.make_async_copy(src_h.at[c, s, 0], sidx_v.at[0],
                                  sisem).wait()
            pltpu.make_async_copy(dst_h.at[c, s, 0], didx_v.at[0],
                                  disem).wait()

        def gather_start(b, par):
            pltpu.async_copy(x_h.at[sidx_v.at[par, b]], rows[b], gsems[b])

        def gather_wait(b):
            pltpu.make_async_copy(x_h.at[sidx_v.at[0, 0]], rows[b],
                                  gsems[b]).wait()

        def scatter_start(b, par):
            pltpu.async_copy(rows[b], acc.at[didx_v.at[par, b]], ssems[b],
                             add=True)

        def scatter_wait(b):
            pltpu.make_async_copy(rows[b], acc.at[didx_v.at[0, 0]],
                                  ssems[b]).wait()

        idx_start(0, 0)
        idx_wait()
        for b in range(NBUF):
            gather_start(b, 0)
        idx_start(jnp.minimum(1, n_rounds - 1), 1)

        def round_body(r, _):
            par = lax.rem(r, 2)
            nxt = 1 - par
            for b in range(NBUF):
                gather_wait(b)
                scatter_start(b, par)
            idx_wait()
            for b in range(NBUF):
                scatter_wait(b)
                gather_start(b, nxt)
            idx_start(jnp.minimum(r + 2, n_rounds - 1), par)
            return 0
        lax.fori_loop(0, n_rounds - 1, round_body, 0)

        lpar = (n_rounds - 1) % 2
        for b in range(NBUF):
            gather_wait(b)
            scatter_start(b, lpar)
        idx_wait()
        for b in range(NBUF):
            scatter_wait(b)

        plsc.subcore_barrier()

        r0 = c * n_nodes + s * rpt
        pltpu.sync_copy(acc.at[pl.ds(s * rpt, rpt)], out_h.at[pl.ds(r0, rpt)])

    return agg_kernel(src5, dst5, x2)


def _tc_pre(x, W, b):
    n, d = x.shape
    h = W.shape[1]
    R = 1000

    def body(x_r, W_r, b_r, out_r):
        out_r[...] = jnp.dot(x_r[...], W_r[...],
                             preferred_element_type=jnp.float32) + b_r[...]

    return pl.pallas_call(
        body,
        grid=(n // R,),
        in_specs=[
            pl.BlockSpec((R, d), lambda i: (i, 0)),
            pl.BlockSpec((d, h), lambda i: (0, 0)),
            pl.BlockSpec((1, h), lambda i: (0, 0)),
        ],
        out_specs=pl.BlockSpec((R, h), lambda i: (i, 0)),
        out_shape=jax.ShapeDtypeStruct((n, h), jnp.float32),
    )(x, W, b.reshape(1, h))


def _tc_post1(pre1, agg1_3, Wn):
    n, h = pre1.shape
    d = Wn.shape[0]
    n_pad = agg1_3.shape[1]
    R = 1000

    def body(pre_r, agg_r, Wn_r, out_r, spl_r):
        a = agg_r[0]
        bb = agg_r[1]
        deg = a[:, d:d + 1] + bb[:, d:d + 1]
        inv = 1.0 / jnp.maximum(deg, 1.0)
        hn = (a[:, :d] + bb[:, :d]) * inv
        out = jnp.maximum(
            pre_r[...] + jnp.dot(hn, Wn_r[...],
                                 preferred_element_type=jnp.float32), 0.0)
        out_r[...] = out
        spl_r[0] = out[:, :h // 2]
        spl_r[1] = out[:, h // 2:]

    return pl.pallas_call(
        body,
        grid=(n // R,),
        in_specs=[
            pl.BlockSpec((R, h), lambda i: (i, 0)),
            pl.BlockSpec((2, R, d + LANES), lambda i: (0, i, 0)),
            pl.BlockSpec((d, h), lambda i: (0, 0)),
        ],
        out_specs=[
            pl.BlockSpec((R, h), lambda i: (i, 0)),
            pl.BlockSpec((2, R, h // 2), lambda i: (0, i, 0)),
        ],
        out_shape=[
            jax.ShapeDtypeStruct((n, h), jnp.float32),
            jax.ShapeDtypeStruct((2, n_pad, h // 2), jnp.float32),
        ],
    )(pre1, agg1_3, Wn)


def _tc_post2(pre2, agg2_3, deg2, Wn2, Wn3, n_pad):
    n, h = pre2.shape
    cdim = Wn3.shape[1]
    R = 1000

    def body(pre_r, agg_r, deg_r, Wn2_r, Wn3_r, h2_r, p_r):
        deg = deg_r[:, 0:1] + deg_r[:, 1:2]
        inv = 1.0 / jnp.maximum(deg, 1.0)
        hn = jnp.concatenate([agg_r[0], agg_r[1]], axis=1) * inv
        h2 = jnp.maximum(
            pre_r[...] + jnp.dot(hn, Wn2_r[...],
                                 preferred_element_type=jnp.float32), 0.0)
        h2_r[...] = h2
        p_r[...] = jnp.dot(h2, Wn3_r[...], preferred_element_type=jnp.float32)

    return pl.pallas_call(
        body,
        grid=(n // R,),
        in_specs=[
            pl.BlockSpec((R, h), lambda i: (i, 0)),
            pl.BlockSpec((2, R, h // 2), lambda i: (0, i, 0)),
            pl.BlockSpec((R, 2), lambda i: (i, 0)),
            pl.BlockSpec((h, h), lambda i: (0, 0)),
            pl.BlockSpec((h, cdim), lambda i: (0, 0)),
        ],
        out_specs=[
            pl.BlockSpec((R, h), lambda i: (i, 0)),
            pl.BlockSpec((R, cdim), lambda i: (i, 0)),
        ],
        out_shape=[
            jax.ShapeDtypeStruct((n, h), jnp.float32),
            jax.ShapeDtypeStruct((n_pad, cdim), jnp.float32),
        ],
    )(pre2, agg2_3, deg2, Wn2, Wn3)


def _tc_final(q, agg3_3, deg2):
    n, cdim = q.shape
    R = 1000

    def body(q_r, agg_r, deg_r, out_r):
        deg = deg_r[:, 0:1] + deg_r[:, 1:2]
        inv = 1.0 / jnp.maximum(deg, 1.0)
        out_r[...] = q_r[...] + (agg_r[0] + agg_r[1]) * inv

    return pl.pallas_call(
        body,
        grid=(n // R,),
        in_specs=[
            pl.BlockSpec((R, cdim), lambda i: (i, 0)),
            pl.BlockSpec((2, R, cdim), lambda i: (0, i, 0)),
            pl.BlockSpec((R, 2), lambda i: (i, 0)),
        ],
        out_specs=pl.BlockSpec((R, cdim), lambda i: (i, 0)),
        out_shape=jax.ShapeDtypeStruct((n, cdim), jnp.float32),
    )(q, agg3_3, deg2)


def kernel(features, edge_index, Ws1, Wn1, b1, Ws2, Wn2, b2, Ws3, Wn3, b3):
    n, d = features.shape
    h = Ws1.shape[1]
    cdim = Ws3.shape[1]
    e = edge_index.shape[1]
    src = edge_index[0]
    dst = edge_index[1]
    align = NS * ZR
    n_pad = ((n + align - 1) // align) * align
    KE, KC, NBUF = 40, 32, 5
    src_e = src.reshape(NC, NS, e // (NC * NS * NBUF * KE), NBUF, KE)
    dst_e = dst.reshape(NC, NS, e // (NC * NS * NBUF * KE), NBUF, KE)
    off = (jnp.arange(NC, dtype=jnp.int32) * n_pad)[:, None]
    src_c = (src[None, :] + off).reshape(
        NC, NS, e // (NS * NBUF * KC), NBUF, KC)
    dst_c = jnp.broadcast_to(dst, (NC, e)).reshape(
        NC, NS, e // (NS * NBUF * KC), NBUF, KC)

    x_aug = jnp.concatenate(
        [jnp.pad(features, ((0, n_pad - n), (0, 0))),
         jnp.ones((n_pad, LANES), jnp.float32)], axis=1)
    agg1 = _sc_aggregate(x_aug, src_e, dst_e, n_pad, e, d + LANES,
                         edge_split=True)
    pre1 = _tc_pre(features, Ws1, b1)
    agg1_3 = agg1.reshape(NC, n_pad, d + LANES)
    deg2 = jnp.concatenate(
        [agg1[:n, d:d + 1], agg1[n_pad:n_pad + n, d:d + 1]], axis=1)
    h1, h1s = _tc_post1(pre1, agg1_3, Wn1)

    agg2_2 = _sc_aggregate(h1s.reshape(NC * n_pad, h // NC), src_c, dst_c,
                           n_pad, e, h // NC, edge_split=False)
    pre2 = _tc_pre(h1, Ws2, b2)
    h2, p3_pad = _tc_post2(pre2, agg2_2.reshape(NC, n_pad, h // NC), deg2,
                           Wn2, Wn3, n_pad)

    agg3 = _sc_aggregate(p3_pad, src_e, dst_e, n_pad, e, cdim,
                         edge_split=True)
    q3 = _tc_pre(h2, Ws3, b3)
    return _tc_final(q3, agg3.reshape(NC, n_pad, cdim), deg2)


# speedup vs baseline: 1.0194x; 1.0194x over previous
import functools

import jax
import jax.numpy as jnp
from jax import lax
from jax.experimental import pallas as pl
from jax.experimental.pallas import tpu as pltpu
from jax.experimental.pallas import tpu_sc as plsc

NC = 2
NS = 16
LANES = 16
ZR = 80


def _sc_aggregate(x2, src5, dst5, n_nodes, n_edges, wc, edge_split):
    K = src5.shape[4]
    NBUF = src5.shape[3]
    n_rounds = src5.shape[2]
    rpt = n_nodes // NS
    assert n_nodes % NS == 0 and rpt % K == 0

    mesh = plsc.VectorSubcoreMesh(core_axis_name="c", subcore_axis_name="s")

    scratch = (
        pltpu.VMEM_SHARED((n_nodes, wc), jnp.float32),
        pltpu.VMEM((2, NBUF, K), jnp.int32),
        pltpu.VMEM((2, NBUF, K), jnp.int32),
        [pltpu.VMEM((K, wc), jnp.float32) for _ in range(NBUF)],
        [pltpu.SemaphoreType.DMA for _ in range(NBUF)],
        [pltpu.SemaphoreType.DMA for _ in range(NBUF)],
        pltpu.SemaphoreType.DMA,
        pltpu.SemaphoreType.DMA,
    )

    @functools.partial(
        pl.kernel,
        out_type=jax.ShapeDtypeStruct((NC * n_nodes, wc), jnp.float32),
        mesh=mesh, scratch_types=scratch,
        compiler_params=pltpu.CompilerParams(use_tc_tiling_on_sc=False))
    def agg_kernel(src_h, dst_h, x_h, out_h,
                   acc, sidx_v, didx_v, rows, gsems, ssems, sisem, disem):
        c = lax.axis_index("c")
        s = lax.axis_index("s")

        zvec = jnp.zeros((LANES,), jnp.float32)

        def fill_zero(r, _):
            for k in range(wc // LANES):
                rows[0][r, pl.ds(k * LANES, LANES)] = zvec
            return 0
        lax.fori_loop(0, K, fill_zero, 0)

        def zero_acc(j, _):
            pltpu.sync_copy(rows[0], acc.at[pl.ds(s * rpt + j * K, K)])
            return 0
        lax.fori_loop(0, rpt // K, zero_acc, 0)

        plsc.subcore_barrier()

        def idx_start(r, par):
            pltpu.async_copy(src_h.at[c, s, r], sidx_v.at[par], sisem)
            pltpu.async_copy(dst_h.at[c, s, r], didx_v.at[par], disem)

        def idx_wait():
            pltpu.make_async_copy(src_h.at[c, s, 0], sidx_v.at[0],
                                  sisem).wait()
            pltpu.make_async_copy(dst_h.at[c, s, 0], didx_v.at[0],
                                  disem).wait()

        def gather_start(b, par):
            pltpu.async_copy(x_h.at[sidx_v.at[par, b]], rows[b], gsems[b])

        def gather_wait(b):
            pltpu.make_async_copy(x_h.at[sidx_v.at[0, 0]], rows[b],
                                  gsems[b]).wait()

        def scatter_start(b, par):
            pltpu.async_copy(rows[b], acc.at[didx_v.at[par, b]], ssems[b],
                             add=True)

        def scatter_wait(b):
            pltpu.make_async_copy(rows[b], acc.at[didx_v.at[0, 0]],
                                  ssems[b]).wait()

        idx_start(0, 0)
        idx_wait()
        for b in range(NBUF):
            gather_start(b, 0)
        idx_start(jnp.minimum(1, n_rounds - 1), 1)

        def round_body(r, _):
            par = lax.rem(r, 2)
            nxt = 1 - par
            for b in range(NBUF):
                gather_wait(b)
                scatter_start(b, par)
            idx_wait()
            for b in range(NBUF):
                scatter_wait(b)
                gather_start(b, nxt)
            idx_start(jnp.minimum(r + 2, n_rounds - 1), par)
            return 0
        lax.fori_loop(0, n_rounds - 1, round_body, 0)

        lpar = (n_rounds - 1) % 2
        for b in range(NBUF):
            gather_wait(b)
            scatter_start(b, lpar)
        idx_wait()
        for b in range(NBUF):
            scatter_wait(b)

        plsc.subcore_barrier()

        r0 = c * n_nodes + s * rpt
        pltpu.sync_copy(acc.at[pl.ds(s * rpt, rpt)], out_h.at[pl.ds(r0, rpt)])

    return agg_kernel(src5, dst5, x2)


def _tc_pre(x, W, b):
    n, d = x.shape
    h = W.shape[1]
    R = 1000

    def body(x_r, W_r, b_r, out_r):
        out_r[...] = jnp.dot(x_r[...], W_r[...],
                             preferred_element_type=jnp.float32) + b_r[...]

    return pl.pallas_call(
        body,
        grid=(n // R,),
        in_specs=[
            pl.BlockSpec((R, d), lambda i: (i, 0)),
            pl.BlockSpec((d, h), lambda i: (0, 0)),
            pl.BlockSpec((1, h), lambda i: (0, 0)),
        ],
        out_specs=pl.BlockSpec((R, h), lambda i: (i, 0)),
        out_shape=jax.ShapeDtypeStruct((n, h), jnp.float32),
    )(x, W, b.reshape(1, h))


def _tc_post1(pre1, agg1_3, Wn):
    n, h = pre1.shape
    d = Wn.shape[0]
    n_pad = agg1_3.shape[1]
    R = 1000

    def body(pre_r, agg_r, Wn_r, out_r, spl_r):
        a = agg_r[0]
        bb = agg_r[1]
        deg = a[:, d:d + 1] + bb[:, d:d + 1]
        inv = 1.0 / jnp.maximum(deg, 1.0)
        hn = (a[:, :d] + bb[:, :d]) * inv
        out = jnp.maximum(
            pre_r[...] + jnp.dot(hn, Wn_r[...],
                                 preferred_element_type=jnp.float32), 0.0)
        out_r[...] = out
        spl_r[0] = out[:, :h // 2]
        spl_r[1] = out[:, h // 2:]

    return pl.pallas_call(
        body,
        grid=(n // R,),
        in_specs=[
            pl.BlockSpec((R, h), lambda i: (i, 0)),
            pl.BlockSpec((2, R, d + LANES), lambda i: (0, i, 0)),
            pl.BlockSpec((d, h), lambda i: (0, 0)),
        ],
        out_specs=[
            pl.BlockSpec((R, h), lambda i: (i, 0)),
            pl.BlockSpec((2, R, h // 2), lambda i: (0, i, 0)),
        ],
        out_shape=[
            jax.ShapeDtypeStruct((n, h), jnp.float32),
            jax.ShapeDtypeStruct((2, n_pad, h // 2), jnp.float32),
        ],
    )(pre1, agg1_3, Wn)


def _tc_post2(pre2, agg2_3, deg2, Wn2, Wn3, n_pad):
    n, h = pre2.shape
    cdim = Wn3.shape[1]
    R = 1000

    def body(pre_r, agg_r, deg_r, Wn2_r, Wn3_r, h2_r, p_r):
        deg = deg_r[:, 0:1] + deg_r[:, 1:2]
        inv = 1.0 / jnp.maximum(deg, 1.0)
        hn = jnp.concatenate([agg_r[0], agg_r[1]], axis=1) * inv
        h2 = jnp.maximum(
            pre_r[...] + jnp.dot(hn, Wn2_r[...],
                                 preferred_element_type=jnp.float32), 0.0)
        h2_r[...] = h2
        p_r[...] = jnp.dot(h2, Wn3_r[...], preferred_element_type=jnp.float32)

    return pl.pallas_call(
        body,
        grid=(n // R,),
        in_specs=[
            pl.BlockSpec((R, h), lambda i: (i, 0)),
            pl.BlockSpec((2, R, h // 2), lambda i: (0, i, 0)),
            pl.BlockSpec((R, 2), lambda i: (i, 0)),
            pl.BlockSpec((h, h), lambda i: (0, 0)),
            pl.BlockSpec((h, cdim), lambda i: (0, 0)),
        ],
        out_specs=[
            pl.BlockSpec((R, h), lambda i: (i, 0)),
            pl.BlockSpec((R, cdim), lambda i: (i, 0)),
        ],
        out_shape=[
            jax.ShapeDtypeStruct((n, h), jnp.float32),
            jax.ShapeDtypeStruct((n_pad, cdim), jnp.float32),
        ],
    )(pre2, agg2_3, deg2, Wn2, Wn3)


def _tc_final(q, agg3_3, deg2):
    n, cdim = q.shape
    R = 1000

    def body(q_r, agg_r, deg_r, out_r):
        deg = deg_r[:, 0:1] + deg_r[:, 1:2]
        inv = 1.0 / jnp.maximum(deg, 1.0)
        out_r[...] = q_r[...] + (agg_r[0] + agg_r[1]) * inv

    return pl.pallas_call(
        body,
        grid=(n // R,),
        in_specs=[
            pl.BlockSpec((R, cdim), lambda i: (i, 0)),
            pl.BlockSpec((2, R, cdim), lambda i: (0, i, 0)),
            pl.BlockSpec((R, 2), lambda i: (i, 0)),
        ],
        out_specs=pl.BlockSpec((R, cdim), lambda i: (i, 0)),
        out_shape=jax.ShapeDtypeStruct((n, cdim), jnp.float32),
    )(q, agg3_3, deg2)


def kernel(features, edge_index, Ws1, Wn1, b1, Ws2, Wn2, b2, Ws3, Wn3, b3):
    n, d = features.shape
    h = Ws1.shape[1]
    cdim = Ws3.shape[1]
    e = edge_index.shape[1]
    src = edge_index[0]
    dst = edge_index[1]
    align = NS * ZR
    n_pad = ((n + align - 1) // align) * align
    K1, K3, KC, NBUF = 40, 80, 40, 5
    def esplit(a, k):
        return a.reshape(NC, NS, e // (NC * NS * NBUF * k), NBUF, k)
    src_e1, dst_e1 = esplit(src, K1), esplit(dst, K1)
    src_e3, dst_e3 = esplit(src, K3), esplit(dst, K3)
    off = (jnp.arange(NC, dtype=jnp.int32) * n_pad)[:, None]
    src_c = (src[None, :] + off).reshape(
        NC, NS, e // (NS * NBUF * KC), NBUF, KC)
    dst_c = jnp.broadcast_to(dst, (NC, e)).reshape(
        NC, NS, e // (NS * NBUF * KC), NBUF, KC)

    x_aug = jnp.concatenate(
        [jnp.pad(features, ((0, n_pad - n), (0, 0))),
         jnp.ones((n_pad, LANES), jnp.float32)], axis=1)
    agg1 = _sc_aggregate(x_aug, src_e1, dst_e1, n_pad, e, d + LANES,
                         edge_split=True)
    pre1 = _tc_pre(features, Ws1, b1)
    agg1_3 = agg1.reshape(NC, n_pad, d + LANES)
    deg2 = jnp.concatenate(
        [agg1[:n, d:d + 1], agg1[n_pad:n_pad + n, d:d + 1]], axis=1)
    h1, h1s = _tc_post1(pre1, agg1_3, Wn1)

    agg2_2 = _sc_aggregate(h1s.reshape(NC * n_pad, h // NC), src_c, dst_c,
                           n_pad, e, h // NC, edge_split=False)
    pre2 = _tc_pre(h1, Ws2, b2)
    h2, p3_pad = _tc_post2(pre2, agg2_2.reshape(NC, n_pad, h // NC), deg2,
                           Wn2, Wn3, n_pad)

    agg3 = _sc_aggregate(p3_pad, src_e3, dst_e3, n_pad, e, cdim,
                         edge_split=True)
    q3 = _tc_pre(h2, Ws3, b3)
    return _tc_final(q3, agg3.reshape(NC, n_pad, cdim), deg2)


# speedup vs baseline: 1.0445x; 1.0247x over previous
import functools

import jax
import jax.numpy as jnp
from jax import lax
from jax.experimental import pallas as pl
from jax.experimental.pallas import tpu as pltpu
from jax.experimental.pallas import tpu_sc as plsc

NC = 2
NS = 16
LANES = 16
ZR = 80


def _sc_aggregate(x2, src5, dst5, n_nodes, n_edges, wc, edge_split):
    K = src5.shape[4]
    NBUF = src5.shape[3]
    n_rounds = src5.shape[2]
    rpt = n_nodes // NS
    assert n_nodes % NS == 0 and rpt % K == 0

    mesh = plsc.VectorSubcoreMesh(core_axis_name="c", subcore_axis_name="s")

    scratch = (
        pltpu.VMEM_SHARED((n_nodes, wc), jnp.float32),
        pltpu.VMEM((2, NBUF, K), jnp.int32),
        pltpu.VMEM((2, NBUF, K), jnp.int32),
        [pltpu.VMEM((K, wc), jnp.float32) for _ in range(NBUF)],
        [pltpu.SemaphoreType.DMA for _ in range(NBUF)],
        [pltpu.SemaphoreType.DMA for _ in range(NBUF)],
        pltpu.SemaphoreType.DMA,
        pltpu.SemaphoreType.DMA,
    )

    @functools.partial(
        pl.kernel,
        out_type=jax.ShapeDtypeStruct((NC * n_nodes, wc), jnp.float32),
        mesh=mesh, scratch_types=scratch,
        compiler_params=pltpu.CompilerParams(use_tc_tiling_on_sc=False))
    def agg_kernel(src_h, dst_h, x_h, out_h,
                   acc, sidx_v, didx_v, rows, gsems, ssems, sisem, disem):
        c = lax.axis_index("c")
        s = lax.axis_index("s")

        def idx_start(r, par):
            pltpu.async_copy(src_h.at[c, s, r], sidx_v.at[par], sisem)
            pltpu.async_copy(dst_h.at[c, s, r], didx_v.at[par], disem)

        def idx_wait():
            pltpu.make_async_copy(src_h.at[c, s, 0], sidx_v.at[0],
                                  sisem).wait()
            pltpu.make_async_copy(dst_h.at[c, s, 0], didx_v.at[0],
                                  disem).wait()

        def gather_start(b, par):
            pltpu.async_copy(x_h.at[sidx_v.at[par, b]], rows[b], gsems[b])

        def gather_wait(b):
            pltpu.make_async_copy(x_h.at[sidx_v.at[0, 0]], rows[b],
                                  gsems[b]).wait()

        def scatter_start(b, par):
            pltpu.async_copy(rows[b], acc.at[didx_v.at[par, b]], ssems[b],
                             add=True)

        def scatter_wait(b):
            pltpu.make_async_copy(rows[b], acc.at[didx_v.at[0, 0]],
                                  ssems[b]).wait()

        idx_start(0, 0)
        zvec = jnp.zeros((LANES,), jnp.float32)

        def fill_zero(r, _):
            for k in range(wc // LANES):
                rows[0][r, pl.ds(k * LANES, LANES)] = zvec
            return 0
        lax.fori_loop(0, K, fill_zero, 0)

        idx_wait()
        for b in range(1, NBUF):
            gather_start(b, 0)
        nz = rpt // K
        for j in range(nz):
            pltpu.async_copy(rows[0], acc.at[pl.ds(s * rpt + j * K, K)],
                             ssems[j % NBUF])
        for j in range(nz):
            pltpu.make_async_copy(rows[0], acc.at[pl.ds(s * rpt + j * K, K)],
                                  ssems[j % NBUF]).wait()
        gather_start(0, 0)
        idx_start(jnp.minimum(1, n_rounds - 1), 1)
        plsc.subcore_barrier()

        def round_body(r, _):
            par = lax.rem(r, 2)
            nxt = 1 - par
            for b in range(NBUF):
                gather_wait(b)
                scatter_start(b, par)
            idx_wait()
            for b in range(NBUF):
                scatter_wait(b)
                gather_start(b, nxt)
            idx_start(jnp.minimum(r + 2, n_rounds - 1), par)
            return 0
        lax.fori_loop(0, n_rounds - 1, round_body, 0)

        lpar = (n_rounds - 1) % 2
        for b in range(NBUF):
            gather_wait(b)
            scatter_start(b, lpar)
        idx_wait()
        for b in range(NBUF):
            scatter_wait(b)

        plsc.subcore_barrier()

        r0 = c * n_nodes + s * rpt
        pltpu.sync_copy(acc.at[pl.ds(s * rpt, rpt)], out_h.at[pl.ds(r0, rpt)])

    return agg_kernel(src5, dst5, x2)


def _tc_pre(x, W, b):
    n, d = x.shape
    h = W.shape[1]
    R = 1000

    def body(x_r, W_r, b_r, out_r):
        out_r[...] = jnp.dot(x_r[...], W_r[...],
                             preferred_element_type=jnp.float32) + b_r[...]

    return pl.pallas_call(
        body,
        grid=(n // R,),
        in_specs=[
            pl.BlockSpec((R, d), lambda i: (i, 0)),
            pl.BlockSpec((d, h), lambda i: (0, 0)),
            pl.BlockSpec((1, h), lambda i: (0, 0)),
        ],
        out_specs=pl.BlockSpec((R, h), lambda i: (i, 0)),
        out_shape=jax.ShapeDtypeStruct((n, h), jnp.float32),
    )(x, W, b.reshape(1, h))


def _tc_post1(pre1, agg1_3, Wn):
    n, h = pre1.shape
    d = Wn.shape[0]
    n_pad = agg1_3.shape[1]
    R = 1000

    def body(pre_r, agg_r, Wn_r, out_r, spl_r):
        a = agg_r[0]
        bb = agg_r[1]
        deg = a[:, d:d + 1] + bb[:, d:d + 1]
        inv = 1.0 / jnp.maximum(deg, 1.0)
        hn = (a[:, :d] + bb[:, :d]) * inv
        out = jnp.maximum(
            pre_r[...] + jnp.dot(hn, Wn_r[...],
                                 preferred_element_type=jnp.float32), 0.0)
        out_r[...] = out
        spl_r[0] = out[:, :h // 2]
        spl_r[1] = out[:, h // 2:]

    return pl.pallas_call(
        body,
        grid=(n // R,),
        in_specs=[
            pl.BlockSpec((R, h), lambda i: (i, 0)),
            pl.BlockSpec((2, R, d + LANES), lambda i: (0, i, 0)),
            pl.BlockSpec((d, h), lambda i: (0, 0)),
        ],
        out_specs=[
            pl.BlockSpec((R, h), lambda i: (i, 0)),
            pl.BlockSpec((2, R, h // 2), lambda i: (0, i, 0)),
        ],
        out_shape=[
            jax.ShapeDtypeStruct((n, h), jnp.float32),
            jax.ShapeDtypeStruct((2, n_pad, h // 2), jnp.float32),
        ],
    )(pre1, agg1_3, Wn)


def _tc_post2(pre2, agg2_3, deg2, Wn2, Wn3, n_pad):
    n, h = pre2.shape
    cdim = Wn3.shape[1]
    R = 1000

    def body(pre_r, agg_r, deg_r, Wn2_r, Wn3_r, h2_r, p_r):
        deg = deg_r[:, 0:1] + deg_r[:, 1:2]
        inv = 1.0 / jnp.maximum(deg, 1.0)
        hn = jnp.concatenate([agg_r[0], agg_r[1]], axis=1) * inv
        h2 = jnp.maximum(
            pre_r[...] + jnp.dot(hn, Wn2_r[...],
                                 preferred_element_type=jnp.float32), 0.0)
        h2_r[...] = h2
        p_r[...] = jnp.dot(h2, Wn3_r[...], preferred_element_type=jnp.float32)

    return pl.pallas_call(
        body,
        grid=(n // R,),
        in_specs=[
            pl.BlockSpec((R, h), lambda i: (i, 0)),
            pl.BlockSpec((2, R, h // 2), lambda i: (0, i, 0)),
            pl.BlockSpec((R, 2), lambda i: (i, 0)),
            pl.BlockSpec((h, h), lambda i: (0, 0)),
            pl.BlockSpec((h, cdim), lambda i: (0, 0)),
        ],
        out_specs=[
            pl.BlockSpec((R, h), lambda i: (i, 0)),
            pl.BlockSpec((R, cdim), lambda i: (i, 0)),
        ],
        out_shape=[
            jax.ShapeDtypeStruct((n, h), jnp.float32),
            jax.ShapeDtypeStruct((n_pad, cdim), jnp.float32),
        ],
    )(pre2, agg2_3, deg2, Wn2, Wn3)


def _tc_final(q, agg3_3, deg2):
    n, cdim = q.shape
    R = 1000

    def body(q_r, agg_r, deg_r, out_r):
        deg = deg_r[:, 0:1] + deg_r[:, 1:2]
        inv = 1.0 / jnp.maximum(deg, 1.0)
        out_r[...] = q_r[...] + (agg_r[0] + agg_r[1]) * inv

    return pl.pallas_call(
        body,
        grid=(n // R,),
        in_specs=[
            pl.BlockSpec((R, cdim), lambda i: (i, 0)),
            pl.BlockSpec((2, R, cdim), lambda i: (0, i, 0)),
            pl.BlockSpec((R, 2), lambda i: (i, 0)),
        ],
        out_specs=pl.BlockSpec((R, cdim), lambda i: (i, 0)),
        out_shape=jax.ShapeDtypeStruct((n, cdim), jnp.float32),
    )(q, agg3_3, deg2)


def kernel(features, edge_index, Ws1, Wn1, b1, Ws2, Wn2, b2, Ws3, Wn3, b3):
    n, d = features.shape
    h = Ws1.shape[1]
    cdim = Ws3.shape[1]
    e = edge_index.shape[1]
    src = edge_index[0]
    dst = edge_index[1]
    align = NS * ZR
    n_pad = ((n + align - 1) // align) * align
    K1, K3, KC, NBUF = 40, 80, 40, 5
    def esplit(a, k):
        return a.reshape(NC, NS, e // (NC * NS * NBUF * k), NBUF, k)
    src_e1, dst_e1 = esplit(src, K1), esplit(dst, K1)
    src_e3, dst_e3 = esplit(src, K3), esplit(dst, K3)
    off = (jnp.arange(NC, dtype=jnp.int32) * n_pad)[:, None]
    src_c = (src[None, :] + off).reshape(
        NC, NS, e // (NS * NBUF * KC), NBUF, KC)
    dst_c = jnp.broadcast_to(dst, (NC, e)).reshape(
        NC, NS, e // (NS * NBUF * KC), NBUF, KC)

    x_aug = jnp.concatenate(
        [jnp.pad(features, ((0, n_pad - n), (0, 0))),
         jnp.ones((n_pad, LANES), jnp.float32)], axis=1)
    agg1 = _sc_aggregate(x_aug, src_e1, dst_e1, n_pad, e, d + LANES,
                         edge_split=True)
    pre1 = _tc_pre(features, Ws1, b1)
    agg1_3 = agg1.reshape(NC, n_pad, d + LANES)
    deg2 = jnp.concatenate(
        [agg1[:n, d:d + 1], agg1[n_pad:n_pad + n, d:d + 1]], axis=1)
    h1, h1s = _tc_post1(pre1, agg1_3, Wn1)

    agg2_2 = _sc_aggregate(h1s.reshape(NC * n_pad, h // NC), src_c, dst_c,
                           n_pad, e, h // NC, edge_split=False)
    pre2 = _tc_pre(h1, Ws2, b2)
    h2, p3_pad = _tc_post2(pre2, agg2_2.reshape(NC, n_pad, h // NC), deg2,
                           Wn2, Wn3, n_pad)

    agg3 = _sc_aggregate(p3_pad, src_e3, dst_e3, n_pad, e, cdim,
                         edge_split=True)
    q3 = _tc_pre(h2, Ws3, b3)
    return _tc_final(q3, agg3.reshape(NC, n_pad, cdim), deg2)
